# baseline (speedup 1.0000x reference)
import functools

import jax
import jax.numpy as jnp
from jax import lax
from jax.experimental import pallas as pl
from jax.experimental.pallas import tpu as pltpu
from jax.experimental.pallas import tpu_sc as plsc

_C = 32
_NBUF = 2
_REP = 16


def _sc_embed(idx_hbm, tab_hbm, out_hbm, idx_v, rows_v, gsem, ssem):
    nc = 2
    wid = lax.axis_index("s") * nc + lax.axis_index("c")
    n_rows = out_hbm.shape[0]
    b_per_w = n_rows // 32
    base = wid * b_per_w
    n_ch = b_per_w // _C

    pltpu.sync_copy(idx_hbm.at[pl.ds(base, b_per_w)], idx_v)

    scat = [None] * n_ch
    for i in range(n_ch):
        b = i % _NBUF
        if i >= _NBUF:
            scat[i - _NBUF].wait()
        g = pltpu.async_copy(
            tab_hbm.at[idx_v.at[pl.ds(i * _C, _C)]], rows_v.at[b], gsem)
        g.wait()
        scat[i] = pltpu.async_copy(
            rows_v.at[b], out_hbm.at[pl.ds(base + i * _C, _C)], ssem)
    for i in range(n_ch - _NBUF, n_ch):
        scat[i].wait()


def kernel(inputs, table):
    B, L = inputs.shape
    H = table.shape[1]
    n = B * L
    tab_rep = jnp.tile(table, (_REP, 1))
    idx = inputs.reshape(n) + 2 * (jnp.arange(n, dtype=inputs.dtype) % _REP)
    mesh = plsc.VectorSubcoreMesh(core_axis_name="c", subcore_axis_name="s")
    k = functools.partial(
        pl.kernel,
        mesh=mesh,
        out_type=jax.ShapeDtypeStruct((n, H), jnp.float32),
        scratch_types=[
            pltpu.VMEM((n // 32,), jnp.int32),
            pltpu.VMEM((_NBUF, _C, H), jnp.float32),
            pltpu.SemaphoreType.DMA,
            pltpu.SemaphoreType.DMA,
        ],
    )(_sc_embed)
    out = k(idx, tab_rep)
    return out.reshape(B, L, H)


# speedup vs baseline: 1.6734x; 1.6734x over previous
import functools

import jax
import jax.numpy as jnp
from jax import lax
from jax.experimental import pallas as pl
from jax.experimental.pallas import tpu as pltpu
from jax.experimental.pallas import tpu_sc as plsc

_C = 32
_NBUF = 2
_REP = 64


def _sc_embed(idx_hbm, tab_hbm, out_hbm, idx_v, rows_v, gsem, ssem):
    nc = 2
    wid = lax.axis_index("s") * nc + lax.axis_index("c")
    n_rows = out_hbm.shape[0]
    b_per_w = n_rows // 32
    base = wid * b_per_w
    n_ch = b_per_w // _C

    pltpu.sync_copy(idx_hbm.at[pl.ds(base, b_per_w)], idx_v)

    scat = [None] * n_ch
    for i in range(n_ch):
        b = i % _NBUF
        if i >= _NBUF:
            scat[i - _NBUF].wait()
        g = pltpu.async_copy(
            tab_hbm.at[idx_v.at[pl.ds(i * _C, _C)]], rows_v.at[b], gsem)
        g.wait()
        scat[i] = pltpu.async_copy(
            rows_v.at[b], out_hbm.at[pl.ds(base + i * _C, _C)], ssem)
    for i in range(n_ch - _NBUF, n_ch):
        scat[i].wait()


def kernel(inputs, table):
    B, L = inputs.shape
    H = table.shape[1]
    n = B * L
    tab_rep = jnp.tile(table, (_REP, 1))
    idx = inputs.reshape(n) + 2 * (jnp.arange(n, dtype=inputs.dtype) % _REP)
    mesh = plsc.VectorSubcoreMesh(core_axis_name="c", subcore_axis_name="s")
    k = functools.partial(
        pl.kernel,
        mesh=mesh,
        out_type=jax.ShapeDtypeStruct((n, H), jnp.float32),
        scratch_types=[
            pltpu.VMEM((n // 32,), jnp.int32),
            pltpu.VMEM((_NBUF, _C, H), jnp.float32),
            pltpu.SemaphoreType.DMA,
            pltpu.SemaphoreType.DMA,
        ],
    )(_sc_embed)
    out = k(idx, tab_rep)
    return out.reshape(B, L, H)


# speedup vs baseline: 2.1739x; 1.2991x over previous
import functools

import jax
import jax.numpy as jnp
from jax import lax
from jax.experimental import pallas as pl
from jax.experimental.pallas import tpu as pltpu
from jax.experimental.pallas import tpu_sc as plsc

_C = 32
_NBUF = 2
_REP = 16


def _sc_embed(idx_hbm, tab_hbm, out_hbm, idx_v, rows_v, gsem, ssem):
    nc = 2
    wid = lax.axis_index("s") * nc + lax.axis_index("c")
    n_rows = out_hbm.shape[0]
    b_per_w = n_rows // 32
    base = wid * b_per_w
    n_ch = b_per_w // _C

    pltpu.sync_copy(idx_hbm.at[pl.ds(base, b_per_w)], idx_v)

    scat = [None] * n_ch
    for i in range(n_ch):
        b = i % _NBUF
        if i >= _NBUF:
            scat[i - _NBUF].wait()
        g = pltpu.async_copy(
            tab_hbm.at[idx_v.at[pl.ds(i * _C, _C)]], rows_v.at[b], gsem)
        g.wait()
        scat[i] = pltpu.async_copy(
            rows_v.at[b], out_hbm.at[pl.ds(base + i * _C, _C)], ssem)
    for i in range(n_ch - _NBUF, n_ch):
        scat[i].wait()


def kernel(inputs, table):
    B, L = inputs.shape
    H = table.shape[1]
    n = B * L
    j = jnp.arange(n, dtype=inputs.dtype)
    b_per_w = n // 32
    slot = (j // b_per_w) * _REP + (j % _REP)
    tab_rep = jnp.tile(table, (32 * _REP, 1))
    idx = inputs.reshape(n) + 2 * slot
    mesh = plsc.VectorSubcoreMesh(core_axis_name="c", subcore_axis_name="s")
    k = functools.partial(
        pl.kernel,
        mesh=mesh,
        out_type=jax.ShapeDtypeStruct((n, H), jnp.float32),
        scratch_types=[
            pltpu.VMEM((n // 32,), jnp.int32),
            pltpu.VMEM((_NBUF, _C, H), jnp.float32),
            pltpu.SemaphoreType.DMA,
            pltpu.SemaphoreType.DMA,
        ],
    )(_sc_embed)
    out = k(idx, tab_rep)
    return out.reshape(B, L, H)


# speedup vs baseline: 2.2266x; 1.0242x over previous
import functools

import jax
import jax.numpy as jnp
from jax import lax
from jax.experimental import pallas as pl
from jax.experimental.pallas import tpu as pltpu
from jax.experimental.pallas import tpu_sc as plsc

_C = 32
_NBUF = 3
_D = 2
_REP = 16


def _sc_embed(idx_hbm, tab_hbm, out_hbm, idx_v, rows_v, *sems):
    gsems = sems[:_NBUF]
    ssems = sems[_NBUF:]
    nc = 2
    wid = lax.axis_index("s") * nc + lax.axis_index("c")
    n_rows = out_hbm.shape[0]
    b_per_w = n_rows // 32
    base = wid * b_per_w
    n_ch = b_per_w // _C

    pltpu.sync_copy(idx_hbm.at[pl.ds(base, b_per_w)], idx_v)

    def start_gather(i):
        b = i % _NBUF
        return pltpu.async_copy(
            tab_hbm.at[idx_v.at[pl.ds(i * _C, _C)]], rows_v.at[b], gsems[b])

    g = [None] * n_ch
    scat = [None] * n_ch
    for i in range(min(_D, n_ch)):
        g[i] = start_gather(i)
    for i in range(n_ch):
        b = i % _NBUF
        g[i].wait()
        scat[i] = pltpu.async_copy(
            rows_v.at[b], out_hbm.at[pl.ds(base + i * _C, _C)], ssems[b])
        nxt = i + _D
        if nxt < n_ch:
            if nxt >= _NBUF:
                scat[nxt - _NBUF].wait()
            g[nxt] = start_gather(nxt)
    for i in range(max(0, n_ch - _NBUF), n_ch):
        scat[i].wait()


def kernel(inputs, table):
    B, L = inputs.shape
    H = table.shape[1]
    n = B * L
    j = jnp.arange(n, dtype=inputs.dtype)
    b_per_w = n // 32
    slot = (j // b_per_w) * _REP + (j % _REP)
    tab_rep = jnp.tile(table, (32 * _REP, 1))
    idx = inputs.reshape(n) + 2 * slot
    mesh = plsc.VectorSubcoreMesh(core_axis_name="c", subcore_axis_name="s")
    k = functools.partial(
        pl.kernel,
        mesh=mesh,
        out_type=jax.ShapeDtypeStruct((n, H), jnp.float32),
        scratch_types=[
            pltpu.VMEM((n // 32,), jnp.int32),
            pltpu.VMEM((_NBUF, _C, H), jnp.float32),
        ] + [pltpu.SemaphoreType.DMA] * (2 * _NBUF),
    )(_sc_embed)
    out = k(idx, tab_rep)
    return out.reshape(B, L, H)
